# Initial kernel scaffold; baseline (speedup 1.0000x reference)
import jax
import jax.numpy as jnp
from jax.experimental import pallas as pl


def kernel(x, edge_index, adj_values, W1, b1, W2, b2, Wc, bc):
    raise NotImplementedError("write your pallas kernel here")


# speedup vs baseline: 5.9326x; 5.9326x over previous
import functools

import jax
import jax.numpy as jnp
from jax import lax
from jax.experimental import pallas as pl
from jax.experimental.pallas import tpu as pltpu
from jax.experimental.pallas import tpu_sc as plsc

N = 10000
E = 320000
NADJ = 11
NFEAT = 128
NHID = 64
NCLASS = 2
D = NADJ * NHID
DP = 768
NHACC = 128

NC = 2
NS = 16
NW = NC * NS
CHUNK = 64
EPT = 10048
NCHUNK = EPT // CHUNK
EPAD = NW * EPT
ROWS_PT = 320
NPAD = NS * ROWS_PT
TRASH = 5100

BM = 1000


def _mm_body(x_ref, w_ref, o_ref):
    o_ref[...] = jnp.dot(x_ref[...], w_ref[...], preferred_element_type=jnp.float32)


def _tc_matmul(x, w):
    m, k = x.shape
    d = w.shape[1]
    return pl.pallas_call(
        _mm_body,
        grid=(m // BM,),
        in_specs=[
            pl.BlockSpec((BM, k), lambda i: (i, 0)),
            pl.BlockSpec((k, d), lambda i: (0, 0)),
        ],
        out_specs=pl.BlockSpec((BM, d), lambda i: (i, 0)),
        out_shape=jax.ShapeDtypeStruct((m, d), jnp.float32),
    )(x, w)


def _comb_body(p_ref, b_ref, w_ref, o_ref):
    h = jnp.maximum((p_ref[0] + p_ref[1] + b_ref[...]) * (1.0 / NADJ), 0.0)
    o_ref[...] = jnp.dot(h, w_ref[...], preferred_element_type=jnp.float32)


def _tc_combine_mm(p, b, w):
    d = w.shape[1]
    return pl.pallas_call(
        _comb_body,
        grid=(N // BM,),
        in_specs=[
            pl.BlockSpec((2, BM, NHID), lambda i: (0, i, 0)),
            pl.BlockSpec((1, NHID), lambda i: (0, 0)),
            pl.BlockSpec((NHID, d), lambda i: (0, 0)),
        ],
        out_specs=pl.BlockSpec((BM, d), lambda i: (i, 0)),
        out_shape=jax.ShapeDtypeStruct((N, d), jnp.float32),
    )(p, b, w)


def _head_body(p_ref, b_ref, wc_ref, bc_ref, o_ref):
    h = jnp.maximum((p_ref[0] + p_ref[1] + b_ref[...]) * (1.0 / NADJ), 0.0)
    logits = jnp.dot(h, wc_ref[...], preferred_element_type=jnp.float32) + bc_ref[...]
    m = jnp.max(logits, axis=1, keepdims=True)
    ex = jnp.exp(logits - m)
    o_ref[...] = (logits - m) - jnp.log(jnp.sum(ex, axis=1, keepdims=True))


def _tc_head(p, b, wc, bc):
    return pl.pallas_call(
        _head_body,
        grid=(N // BM,),
        in_specs=[
            pl.BlockSpec((2, BM, NHID), lambda i: (0, i, 0)),
            pl.BlockSpec((1, NHID), lambda i: (0, 0)),
            pl.BlockSpec((NHID, NCLASS), lambda i: (0, 0)),
            pl.BlockSpec((1, NCLASS), lambda i: (0, 0)),
        ],
        out_specs=pl.BlockSpec((BM, NCLASS), lambda i: (i, 0)),
        out_shape=jax.ShapeDtypeStruct((N, NCLASS), jnp.float32),
    )(p, b, wc, bc)


def _sc_body(S_hbm, src_hbm, dst_hbm, adj_hbm, out_hbm,
             src_v, dst_v, adj_v, rows_v, ebuf_v, obuf_v, ie_v, io_v, acc_sh, sem):
    c = lax.axis_index("c")
    s = lax.axis_index("s")
    wid = c * NS + s

    def zbuf_body(e, _):
        for g in range(NHACC // 16):
            ebuf_v[e, pl.ds(g * 16, 16)] = jnp.zeros((16,), jnp.float32)
            obuf_v[e, pl.ds(g * 16, 16)] = jnp.zeros((16,), jnp.float32)
        return 0

    lax.fori_loop(0, CHUNK, zbuf_body, 0)

    def zacc_body(r, _):
        pltpu.sync_copy(ebuf_v, acc_sh.at[pl.ds(s * ROWS_PT + r * CHUNK, CHUNK)])
        return 0

    lax.fori_loop(0, ROWS_PT // CHUNK, zacc_body, 0)
    plsc.subcore_barrier()

    def chunk_body(k, _):
        base = pl.multiple_of(wid * EPT + k * CHUNK, 8)
        pltpu.sync_copy(src_hbm.at[pl.ds(base, CHUNK)], src_v)
        pltpu.sync_copy(dst_hbm.at[pl.ds(base, CHUNK)], dst_v)
        pltpu.sync_copy(adj_hbm.at[pl.ds(base, CHUNK)], adj_v)
        pltpu.async_copy(S_hbm.at[src_v], rows_v, sem).wait()

        def idx_body(q, _):
            d = dst_v[pl.ds(q * 16, 16)]
            h = lax.shift_right_logical(d, 1)
            odd = lax.rem(d, 2) == 1
            ie_v[pl.ds(q * 16, 16)] = jnp.where(odd, TRASH, h)
            io_v[pl.ds(q * 16, 16)] = jnp.where(odd, h, TRASH)
            return 0

        lax.fori_loop(0, CHUNK // 16, idx_body, 0)

        def edge_body(e, _):
            av = adj_v[e, :]
            for g in range(NHID // 16):
                acc = av[0] * rows_v[e, pl.ds(g * 16, 16)]
                for i in range(1, NADJ):
                    acc = acc + av[i] * rows_v[e, pl.ds(i * NHID + g * 16, 16)]
                ebuf_v[e, pl.ds(g * 16, 16)] = acc
                obuf_v[e, pl.ds(64 + g * 16, 16)] = acc
            return 0

        lax.fori_loop(0, CHUNK, edge_body, 0)
        pltpu.sync_copy(ebuf_v, acc_sh.at[ie_v], add=True)
        pltpu.sync_copy(obuf_v, acc_sh.at[io_v], add=True)
        return 0

    lax.fori_loop(0, NCHUNK, chunk_body, 0)
    plsc.subcore_barrier()

    pltpu.sync_copy(acc_sh.at[pl.ds(s * ROWS_PT, ROWS_PT)],
                    out_hbm.at[c, pl.ds(s * ROWS_PT, ROWS_PT)])


_sc_spmm = functools.partial(
    pl.kernel,
    out_type=jax.ShapeDtypeStruct((NC, NPAD, NHACC), jnp.float32),
    mesh=plsc.VectorSubcoreMesh(core_axis_name="c", subcore_axis_name="s"),
    scratch_types=[
        pltpu.VMEM((CHUNK,), jnp.int32),
        pltpu.VMEM((CHUNK,), jnp.int32),
        pltpu.VMEM((CHUNK, 16), jnp.float32),
        pltpu.VMEM((CHUNK, DP), jnp.float32),
        pltpu.VMEM((CHUNK, NHACC), jnp.float32),
        pltpu.VMEM((CHUNK, NHACC), jnp.float32),
        pltpu.VMEM((CHUNK,), jnp.int32),
        pltpu.VMEM((CHUNK,), jnp.int32),
        pltpu.VMEM_SHARED((NPAD, NHACC), jnp.float32),
        pltpu.SemaphoreType.DMA,
    ],
)(_sc_body)


def _pad_edges(a, fill):
    a = a.reshape((NW, E // NW) + a.shape[1:])
    pad = [(0, 0), (0, EPT - E // NW)] + [(0, 0)] * (a.ndim - 2)
    return jnp.pad(a, pad, constant_values=fill).reshape((EPAD,) + a.shape[2:])


def kernel(x, edge_index, adj_values, W1, b1, W2, b2, Wc, bc):
    src = _pad_edges(edge_index[0], 0)
    dst = _pad_edges(edge_index[1], 2 * TRASH)
    adjT = _pad_edges(jnp.pad(adj_values.T, ((0, 0), (0, 16 - NADJ))), 0)
    W1c = jnp.pad(W1.transpose(1, 0, 2).reshape(NFEAT, D), ((0, 0), (0, DP - D)))
    W2c = jnp.pad(W2.transpose(1, 0, 2).reshape(NHID, D), ((0, 0), (0, DP - D)))
    b1s = b1.sum(axis=0).reshape(1, NHID)
    b2s = b2.sum(axis=0).reshape(1, NHID)

    S1 = _tc_matmul(x, W1c)
    P1 = _sc_spmm(S1, src, dst, adjT).reshape(NC, 2 * NPAD, NHID)[:, :N]
    S2 = _tc_combine_mm(P1, b1s, W2c)
    P2 = _sc_spmm(S2, src, dst, adjT).reshape(NC, 2 * NPAD, NHID)[:, :N]
    return _tc_head(P2, b2s, Wc, bc.reshape(1, NCLASS))
